# baseline (speedup 1.0000x reference)
import functools

import jax
import jax.numpy as jnp
from jax import lax
from jax.experimental import pallas as pl
from jax.experimental.pallas import tpu as pltpu
from jax.experimental.pallas import tpu_sc as plsc


def _sc_gather(table, idx_flat):
    n_idx = idx_flat.shape[0]
    d = table.shape[1]
    info = plsc.get_sparse_core_info()
    nw = info.num_cores * info.num_subcores
    b_per_w = n_idx // nw
    ch = max(c for c in range(1, 129) if b_per_w % c == 0)
    n_ch = b_per_w // ch

    idx3 = idx_flat.reshape(nw, n_ch, ch)
    mesh = plsc.VectorSubcoreMesh(core_axis_name="c", subcore_axis_name="s")

    @functools.partial(
        pl.kernel,
        mesh=mesh,
        out_type=jax.ShapeDtypeStruct((nw, n_ch, ch, d), jnp.float32),
        scratch_types=[
            pltpu.VMEM((n_ch, ch), jnp.int32),
            pltpu.VMEM((n_ch, ch, d), jnp.float32),
            pltpu.SemaphoreType.DMA((n_ch,)),
            pltpu.SemaphoreType.DMA((n_ch,)),
        ],
    )
    def gather_kernel(table_hbm, idx_hbm, out_hbm, idx_v, rows_v, gsem,
                      osem):
        wid = lax.axis_index("s") * info.num_cores + lax.axis_index("c")
        pltpu.sync_copy(idx_hbm.at[wid], idx_v)
        copies = [
            pltpu.async_copy(table_hbm.at[idx_v.at[j]], rows_v.at[j],
                             gsem.at[j])
            for j in range(n_ch)
        ]
        outs = []
        for j, c in enumerate(copies):
            c.wait()
            outs.append(pltpu.async_copy(rows_v.at[j], out_hbm.at[wid].at[j],
                                         osem.at[j]))
        for o in outs:
            o.wait()

    out = gather_kernel(table, idx3)
    return out.reshape(n_idx, d)


def _gru_body(T, H, cap_ref, len_ref, wih_ref, whh_ref, bih_ref,
              bhh_ref, out_ref):
    wih = wih_ref[...]
    whh = whh_ref[...]
    bih = bih_ref[...]
    bhh = bhh_ref[...]
    lens = len_ref[...]
    bb = cap_ref.shape[1]

    h = jnp.zeros((bb, H), jnp.float32)
    acc = jnp.full((bb, H), jnp.finfo(jnp.float32).min, jnp.float32)
    dn = (((1,), (1,)), ((), ()))
    wih_b = wih.astype(jnp.bfloat16)
    whh_b = whh.astype(jnp.bfloat16)
    brz = bih[:, :2 * H] + bhh[:, :2 * H]
    bin_ = bih[:, 2 * H:]
    bhn = bhh[:, 2 * H:]
    for t in range(T):
        xt = cap_ref[t].astype(jnp.bfloat16)
        gi = lax.dot_general(xt, wih_b, dn,
                             preferred_element_type=jnp.float32)
        gh = lax.dot_general(h.astype(jnp.bfloat16), whh_b, dn,
                             preferred_element_type=jnp.float32)
        rz = jax.nn.sigmoid(gi[:, :2 * H] + gh[:, :2 * H] + brz)
        r = rz[:, :H]
        z = rz[:, H:]
        n = jnp.tanh(gi[:, 2 * H:] + bin_ + r * (gh[:, 2 * H:] + bhn))
        h = n + z * (h - n)
        acc = jnp.where(lens > t, jnp.maximum(acc, h), acc)
    out_ref[...] = acc


def _gru_maxpool(cap_t, lengths, W_ih, W_hh, b_ih, b_hh, block_b=256,
                 interpret=False):
    T, B, D = cap_t.shape
    H = W_hh.shape[1]
    nb = B // block_b
    lens2 = lengths.astype(jnp.int32).reshape(B, 1)
    bih2 = b_ih.reshape(1, 3 * H)
    bhh2 = b_hh.reshape(1, 3 * H)

    return pl.pallas_call(
        functools.partial(_gru_body, T, H),
        grid=(nb,),
        in_specs=[
            pl.BlockSpec((T, block_b, D), lambda b: (0, b, 0)),
            pl.BlockSpec((block_b, 1), lambda b: (b, 0)),
            pl.BlockSpec((3 * H, D), lambda b: (0, 0)),
            pl.BlockSpec((3 * H, H), lambda b: (0, 0)),
            pl.BlockSpec((1, 3 * H), lambda b: (0, 0)),
            pl.BlockSpec((1, 3 * H), lambda b: (0, 0)),
        ],
        out_specs=pl.BlockSpec((block_b, H), lambda b: (b, 0)),
        out_shape=jax.ShapeDtypeStruct((B, H), jnp.float32),
        interpret=interpret,
    )(cap_t, lens2, W_ih, W_hh, bih2, bhh2)


def kernel(x, lengths, embed_table, W_ih, W_hh, b_ih, b_hh):
    B, T = x.shape
    D = embed_table.shape[1]
    idx_t = jnp.transpose(x).reshape(T * B).astype(jnp.int32)
    cap_t = _sc_gather(embed_table, idx_t).reshape(T, B, D)
    cap_emb = jnp.transpose(cap_t, (1, 0, 2))
    outputs = _gru_maxpool(cap_t, lengths, W_ih, W_hh, b_ih, b_hh)
    return (outputs, cap_emb)


# speedup vs baseline: 1.0037x; 1.0037x over previous
import functools

import jax
import jax.numpy as jnp
from jax import lax
from jax.experimental import pallas as pl
from jax.experimental.pallas import tpu as pltpu
from jax.experimental.pallas import tpu_sc as plsc


def _sc_gather(table, idx_flat):
    n_idx = idx_flat.shape[0]
    d = table.shape[1]
    info = plsc.get_sparse_core_info()
    nw = info.num_cores * info.num_subcores
    b_per_w = n_idx // nw
    ch = max(c for c in range(1, 129) if b_per_w % c == 0)
    n_ch = b_per_w // ch

    idx3 = idx_flat.reshape(nw, n_ch, ch)
    mesh = plsc.VectorSubcoreMesh(core_axis_name="c", subcore_axis_name="s")

    @functools.partial(
        pl.kernel,
        mesh=mesh,
        out_type=jax.ShapeDtypeStruct((nw, n_ch, ch, d), jnp.float32),
        scratch_types=[
            pltpu.VMEM((n_ch, ch), jnp.int32),
            pltpu.VMEM((n_ch, ch, d), jnp.float32),
            pltpu.SemaphoreType.DMA,
        ],
    )
    def gather_kernel(table_hbm, idx_hbm, out_hbm, idx_v, rows_v, sem):
        wid = lax.axis_index("s") * info.num_cores + lax.axis_index("c")
        pltpu.sync_copy(idx_hbm.at[wid], idx_v)
        copies = [
            pltpu.async_copy(table_hbm.at[idx_v.at[j]], rows_v.at[j], sem)
            for j in range(n_ch)
        ]
        for c in copies:
            c.wait()
        pltpu.sync_copy(rows_v, out_hbm.at[wid])

    out = gather_kernel(table, idx3)
    return out.reshape(n_idx, d)


def _gru_body(T, H, cap_ref, len_ref, wih_ref, whh_ref, bih_ref,
              bhh_ref, out_ref):
    wih = wih_ref[...]
    whh = whh_ref[...]
    bih = bih_ref[...]
    bhh = bhh_ref[...]
    lens = len_ref[...]
    bb = cap_ref.shape[1]

    h = jnp.zeros((bb, H), jnp.float32)
    acc = jnp.full((bb, H), jnp.finfo(jnp.float32).min, jnp.float32)
    dn = (((1,), (1,)), ((), ()))
    wih_b = wih.astype(jnp.bfloat16)
    whh_b = whh.astype(jnp.bfloat16)
    brz = bih[:, :2 * H] + bhh[:, :2 * H]
    bin_ = bih[:, 2 * H:]
    bhn = bhh[:, 2 * H:]
    for t in range(T):
        xt = cap_ref[t].astype(jnp.bfloat16)
        gi = lax.dot_general(xt, wih_b, dn,
                             preferred_element_type=jnp.float32)
        gh = lax.dot_general(h.astype(jnp.bfloat16), whh_b, dn,
                             preferred_element_type=jnp.float32)
        rz = jax.nn.sigmoid(gi[:, :2 * H] + gh[:, :2 * H] + brz)
        r = rz[:, :H]
        z = rz[:, H:]
        n = jnp.tanh(gi[:, 2 * H:] + bin_ + r * (gh[:, 2 * H:] + bhn))
        h = n + z * (h - n)
        acc = jnp.where(lens > t, jnp.maximum(acc, h), acc)
    out_ref[...] = acc


def _gru_maxpool(cap_t, lengths, W_ih, W_hh, b_ih, b_hh, block_b=256,
                 interpret=False):
    T, B, D = cap_t.shape
    H = W_hh.shape[1]
    nb = B // block_b
    lens2 = lengths.astype(jnp.int32).reshape(B, 1)
    bih2 = b_ih.reshape(1, 3 * H)
    bhh2 = b_hh.reshape(1, 3 * H)

    return pl.pallas_call(
        functools.partial(_gru_body, T, H),
        grid=(nb,),
        in_specs=[
            pl.BlockSpec((T, block_b, D), lambda b: (0, b, 0)),
            pl.BlockSpec((block_b, 1), lambda b: (b, 0)),
            pl.BlockSpec((3 * H, D), lambda b: (0, 0)),
            pl.BlockSpec((3 * H, H), lambda b: (0, 0)),
            pl.BlockSpec((1, 3 * H), lambda b: (0, 0)),
            pl.BlockSpec((1, 3 * H), lambda b: (0, 0)),
        ],
        out_specs=pl.BlockSpec((block_b, H), lambda b: (b, 0)),
        out_shape=jax.ShapeDtypeStruct((B, H), jnp.float32),
        interpret=interpret,
    )(cap_t, lens2, W_ih, W_hh, bih2, bhh2)


def kernel(x, lengths, embed_table, W_ih, W_hh, b_ih, b_hh):
    B, T = x.shape
    D = embed_table.shape[1]
    idx_t = jnp.transpose(x).reshape(T * B).astype(jnp.int32)
    cap_t = _sc_gather(embed_table, idx_t).reshape(T, B, D)
    cap_emb = jnp.transpose(cap_t, (1, 0, 2))
    outputs = _gru_maxpool(cap_t, lengths, W_ih, W_hh, b_ih, b_hh)
    return (outputs, cap_emb)
